# Initial kernel scaffold; baseline (speedup 1.0000x reference)
import jax
import jax.numpy as jnp
from jax.experimental import pallas as pl


def kernel(x, type_embedding):
    raise NotImplementedError("write your pallas kernel here")


# speedup vs baseline: 14.0553x; 14.0553x over previous
import functools

import jax
import jax.numpy as jnp
import numpy as np
from jax.experimental import pallas as pl
from jax.experimental.pallas import tpu as pltpu

_MOVE_DIM = 11
_G = 24
_NT = 19
_ED = 8
_IN = _G * _MOVE_DIM
_OH = _G * _NT
_OUT = _G * (_MOVE_DIM - 1 + _ED)


def _build_static():
    P = np.zeros((_IN, _OUT), np.float32)
    R = np.zeros((_IN, _OH), np.float32)
    for g in range(_G):
        for k in range(_MOVE_DIM - 1):
            P[g * _MOVE_DIM + 1 + k, g * 18 + k] = 1.0
        for t in range(_NT):
            R[g * _MOVE_DIM, g * _NT + t] = 1.0
    tpat = (np.arange(_OH) % _NT).astype(np.float32).reshape(1, _OH)
    return jnp.asarray(P), jnp.asarray(R), jnp.asarray(tpat)


_Pc, _Rc, _TPATc = _build_static()

_ER = np.zeros((_G, _NT, _ED), np.int32)
_EC = np.zeros((_G, _NT, _ED), np.int32)
for _g in range(_G):
    for _t in range(_NT):
        for _c in range(_ED):
            _ER[_g, _t, _c] = _g * _NT + _t
            _EC[_g, _t, _c] = _g * 18 + 10 + _c
_ERj = jnp.asarray(_ER)
_ECj = jnp.asarray(_EC)


def _body(x_ref, e_ref, p_ref, r_ref, tp_ref, o_ref):
    xb = x_ref[...]
    xf = jnp.floor(xb)
    sel = jax.lax.dot(xf, r_ref[...], preferred_element_type=jnp.float32)
    oh = (sel == tp_ref[...]).astype(jnp.float32)
    o_ref[...] = (
        jax.lax.dot(xb, p_ref[...], preferred_element_type=jnp.float32)
        + jax.lax.dot(oh, e_ref[...], preferred_element_type=jnp.float32)
    )


@jax.jit
def kernel(x, type_embedding):
    b, s = x.shape[0], x.shape[1]
    n = b * s
    x2 = x.reshape(n, _IN)

    E = jnp.zeros((_OH, _OUT), jnp.float32).at[_ERj, _ECj].set(
        jnp.broadcast_to(type_embedding[None, :, :], (_G, _NT, _ED))
    )

    blk = 2048
    while n % blk != 0:
        blk //= 2
    grid = (n // blk,)

    out = pl.pallas_call(
        _body,
        grid=grid,
        in_specs=[
            pl.BlockSpec((blk, _IN), lambda i: (i, 0)),
            pl.BlockSpec((_OH, _OUT), lambda i: (0, 0)),
            pl.BlockSpec((_IN, _OUT), lambda i: (0, 0)),
            pl.BlockSpec((_IN, _OH), lambda i: (0, 0)),
            pl.BlockSpec((1, _OH), lambda i: (0, 0)),
        ],
        out_specs=pl.BlockSpec((blk, _OUT), lambda i: (i, 0)),
        out_shape=jax.ShapeDtypeStruct((n, _OUT), jnp.float32),
    )(x2, E, _Pc, _Rc, _TPATc)

    return out.reshape(b, s, _OUT)
